# baseline (speedup 1.0000x reference)
import functools

import jax
import jax.numpy as jnp
from jax import lax
from jax.experimental import pallas as pl
from jax.experimental.pallas import tpu as pltpu
from jax.experimental.pallas import tpu_sc as plsc

N_AIDS = 1000000
EMBED_DIM = 64
BATCH = 16384

NC = 2
NS = 16
L = 16
NW = NC * NS
BPW = BATCH // NW
CH = 128
NCHUNK = BPW // CH

CB = 28
COLS = CB * L
NCH_MAIN = N_AIDS // COLS
TAIL_COL = NCH_MAIN * COLS
TAIL_N = N_AIDS - TAIL_COL
NG = 35


def _transpose_blocks(a_v, b_v, lanes, nblk):
    def jbody(j, carry):
        rows = j * L + lanes
        for d in range(EMBED_DIM):
            v = a_v[d, pl.ds(j * L, L)]
            plsc.store_scatter(b_v, [rows, jnp.full((L,), d, jnp.int32)], v)
        return carry
    lax.fori_loop(0, nblk, jbody, 0)


def _tr_kernel(embt_hbm, out_hbm, a0, a1, b0, b1, rs0, rs1, ws0, ws1):
    wid = lax.axis_index("s") * NC + lax.axis_index("c")
    lanes = lax.iota(jnp.int32, L)
    A, B, RS, WS = (a0, a1), (b0, b1), (rs0, rs1), (ws0, ws1)

    def rd(c, u):
        return pltpu.make_async_copy(
            embt_hbm.at[:, pl.ds(c * COLS, COLS)], A[u], RS[u])

    def wr(c, u):
        return pltpu.make_async_copy(
            B[u], out_hbm.at[pl.ds(c * COLS, COLS), :], WS[u])

    for u in range(2):
        c = wid + NW * u

        @pl.when(c < NCH_MAIN)
        def _(c=c, u=u):
            rd(c, u).start()

    def gbody(g, carry):
        for u in range(2):
            i = 2 * g + u
            c = wid + NW * i

            @pl.when(c < NCH_MAIN)
            def _(c=c, u=u, g=g):
                rd(c, u).wait()

                @pl.when(g >= 1)
                def _():
                    wr(c - 2 * NW, u).wait()

                _transpose_blocks(A[u], B[u], lanes, CB)
                wr(c, u).start()

                @pl.when(c + 2 * NW < NCH_MAIN)
                def _():
                    rd(c + 2 * NW, u).start()
        return carry

    lax.fori_loop(0, NG, gbody, 0)

    k_last = (NCH_MAIN - wid + NW - 1) // NW - 1
    for u in range(2):
        li = k_last - ((k_last - u) % 2)
        wr(wid + NW * li, u).wait()

    @pl.when(wid == NW - 1)
    def _():
        pltpu.sync_copy(embt_hbm.at[:, pl.ds(TAIL_COL, TAIL_N)],
                        a0.at[:, pl.ds(0, TAIL_N)])
        _transpose_blocks(a0, b0, lanes, TAIL_N // L)
        pltpu.sync_copy(b0.at[pl.ds(0, TAIL_N), :],
                        out_hbm.at[pl.ds(TAIL_COL, TAIL_N), :])


def _dot_kernel(aid1_hbm, aid2_hbm, emb_hbm, out_hbm,
                idx1_v, idx2_v, rows1_v, rows2_v, out_v, sem):
    wid = lax.axis_index("s") * NC + lax.axis_index("c")
    base = wid * BPW

    pltpu.sync_copy(aid1_hbm.at[pl.ds(base, BPW)], idx1_v)
    pltpu.sync_copy(aid2_hbm.at[pl.ds(base, BPW)], idx2_v)

    copies = []
    for c in range(NCHUNK):
        copies.append(pltpu.async_copy(
            emb_hbm.at[idx1_v.at[pl.ds(c * CH, CH)]],
            rows1_v.at[pl.ds(c * CH, CH)], sem))
        copies.append(pltpu.async_copy(
            emb_hbm.at[idx2_v.at[pl.ds(c * CH, CH)]],
            rows2_v.at[pl.ds(c * CH, CH)], sem))
    for cp in copies:
        cp.wait()

    lanes = lax.iota(jnp.int32, L)

    def block_body(c, carry):
        rb = c * L
        row_ids = rb + lanes
        acc = jnp.zeros((L,), jnp.float32)
        for d in range(EMBED_DIM):
            col = jnp.full((L,), d, jnp.int32)
            a = plsc.load_gather(rows1_v, [row_ids, col])
            b = plsc.load_gather(rows2_v, [row_ids, col])
            acc = acc + a * b
        out_v[pl.ds(rb, L)] = acc
        return carry

    lax.fori_loop(0, BPW // L, block_body, 0)

    pltpu.sync_copy(out_v, out_hbm.at[pl.ds(base, BPW)])


@jax.jit
def _run(aid1, aid2, embeddings):
    mesh = plsc.VectorSubcoreMesh(core_axis_name="c", subcore_axis_name="s")
    params = pltpu.CompilerParams(
        needs_layout_passes=False, use_tc_tiling_on_sc=False)

    embt = embeddings.T
    emb_rm = functools.partial(
        pl.kernel,
        mesh=mesh,
        compiler_params=params,
        out_type=jax.ShapeDtypeStruct((N_AIDS, EMBED_DIM), jnp.float32),
        scratch_types=[
            pltpu.VMEM((EMBED_DIM, COLS), jnp.float32),
            pltpu.VMEM((EMBED_DIM, COLS), jnp.float32),
            pltpu.VMEM((COLS, EMBED_DIM), jnp.float32),
            pltpu.VMEM((COLS, EMBED_DIM), jnp.float32),
            pltpu.SemaphoreType.DMA,
            pltpu.SemaphoreType.DMA,
            pltpu.SemaphoreType.DMA,
            pltpu.SemaphoreType.DMA,
        ],
    )(_tr_kernel)(embt)

    f = functools.partial(
        pl.kernel,
        mesh=mesh,
        compiler_params=params,
        out_type=jax.ShapeDtypeStruct((BATCH,), jnp.float32),
        scratch_types=[
            pltpu.VMEM((BPW,), jnp.int32),
            pltpu.VMEM((BPW,), jnp.int32),
            pltpu.VMEM((BPW, EMBED_DIM), jnp.float32),
            pltpu.VMEM((BPW, EMBED_DIM), jnp.float32),
            pltpu.VMEM((BPW,), jnp.float32),
            pltpu.SemaphoreType.DMA,
        ],
    )(_dot_kernel)
    return f(aid1, aid2, emb_rm)


def kernel(aid1, aid2, embeddings):
    return _run(aid1.astype(jnp.int32), aid2.astype(jnp.int32), embeddings)


# speedup vs baseline: 1.1022x; 1.1022x over previous
import functools

import jax
import jax.numpy as jnp
from jax import lax
from jax.experimental import pallas as pl
from jax.experimental.pallas import tpu as pltpu
from jax.experimental.pallas import tpu_sc as plsc

N_AIDS = 1000000
EMBED_DIM = 64
BATCH = 16384

NC = 2
NS = 16
L = 16
NW = NC * NS
BPW = BATCH // NW
CH = 128
NCHUNK = BPW // CH

CB = 28
COLS = CB * L
NCH_MAIN = N_AIDS // COLS
TAIL_COL = NCH_MAIN * COLS
TAIL_N = N_AIDS - TAIL_COL
NG = 35


def _transpose_blocks(a_v, b_v, lanes, nblk):
    def jbody(j, carry):
        rows = j * L + lanes
        for d in range(EMBED_DIM):
            v = a_v[d, pl.ds(j * L, L)]
            plsc.store_scatter(b_v, [rows, jnp.full((L,), d, jnp.int32)], v)
        return carry
    lax.fori_loop(0, nblk, jbody, 0)


def _tr_kernel(embt_hbm, out_hbm, a0, a1, b0, b1, rs0, rs1, ws0, ws1):
    wid = lax.axis_index("s") * NC + lax.axis_index("c")
    lanes = lax.iota(jnp.int32, L)
    A, B, RS, WS = (a0, a1), (b0, b1), (rs0, rs1), (ws0, ws1)

    def rd(c, u):
        return pltpu.make_async_copy(
            embt_hbm.at[:, pl.ds(c * COLS, COLS)], A[u], RS[u])

    def wr(c, u):
        return pltpu.make_async_copy(
            B[u].at[:, pl.ds(0, EMBED_DIM)],
            out_hbm.at[pl.ds(c * COLS, COLS), :], WS[u])

    for u in range(2):
        c = wid + NW * u

        @pl.when(c < NCH_MAIN)
        def _(c=c, u=u):
            rd(c, u).start()

    def gbody(g, carry):
        for u in range(2):
            i = 2 * g + u
            c = wid + NW * i

            @pl.when(c < NCH_MAIN)
            def _(c=c, u=u, g=g):
                rd(c, u).wait()

                @pl.when(g >= 1)
                def _():
                    wr(c - 2 * NW, u).wait()

                _transpose_blocks(A[u], B[u], lanes, CB)
                wr(c, u).start()

                @pl.when(c + 2 * NW < NCH_MAIN)
                def _():
                    rd(c + 2 * NW, u).start()
        return carry

    lax.fori_loop(0, NG, gbody, 0)

    k_last = (NCH_MAIN - wid + NW - 1) // NW - 1
    for u in range(2):
        li = k_last - ((k_last - u) % 2)
        wr(wid + NW * li, u).wait()

    @pl.when(wid == NW - 1)
    def _():
        pltpu.sync_copy(embt_hbm.at[:, pl.ds(TAIL_COL, TAIL_N)],
                        a0.at[:, pl.ds(0, TAIL_N)])
        _transpose_blocks(a0, b0, lanes, TAIL_N // L)
        pltpu.sync_copy(b0.at[pl.ds(0, TAIL_N), pl.ds(0, EMBED_DIM)],
                        out_hbm.at[pl.ds(TAIL_COL, TAIL_N), :])


def _dot_kernel(aid1_hbm, aid2_hbm, emb_hbm, out_hbm,
                idx1_v, idx2_v, rows1_v, rows2_v, out_v, sem):
    wid = lax.axis_index("s") * NC + lax.axis_index("c")
    base = wid * BPW

    pltpu.sync_copy(aid1_hbm.at[pl.ds(base, BPW)], idx1_v)
    pltpu.sync_copy(aid2_hbm.at[pl.ds(base, BPW)], idx2_v)

    copies = []
    for c in range(NCHUNK):
        copies.append(pltpu.async_copy(
            emb_hbm.at[idx1_v.at[pl.ds(c * CH, CH)]],
            rows1_v.at[pl.ds(c * CH, CH)], sem))
        copies.append(pltpu.async_copy(
            emb_hbm.at[idx2_v.at[pl.ds(c * CH, CH)]],
            rows2_v.at[pl.ds(c * CH, CH)], sem))
    for cp in copies:
        cp.wait()

    lanes = lax.iota(jnp.int32, L)

    def block_body(c, carry):
        rb = c * L
        row_ids = rb + lanes
        acc = jnp.zeros((L,), jnp.float32)
        for d in range(EMBED_DIM):
            col = jnp.full((L,), d, jnp.int32)
            a = plsc.load_gather(rows1_v, [row_ids, col])
            b = plsc.load_gather(rows2_v, [row_ids, col])
            acc = acc + a * b
        out_v[pl.ds(rb, L)] = acc
        return carry

    lax.fori_loop(0, BPW // L, block_body, 0)

    pltpu.sync_copy(out_v, out_hbm.at[pl.ds(base, BPW)])


@jax.jit
def _run(aid1, aid2, embeddings):
    mesh = plsc.VectorSubcoreMesh(core_axis_name="c", subcore_axis_name="s")
    params = pltpu.CompilerParams(
        needs_layout_passes=False, use_tc_tiling_on_sc=False)

    embt = embeddings.T
    emb_rm = functools.partial(
        pl.kernel,
        mesh=mesh,
        compiler_params=params,
        out_type=jax.ShapeDtypeStruct((N_AIDS, EMBED_DIM), jnp.float32),
        scratch_types=[
            pltpu.VMEM((EMBED_DIM, COLS), jnp.float32),
            pltpu.VMEM((EMBED_DIM, COLS), jnp.float32),
            pltpu.VMEM((COLS, EMBED_DIM + 1), jnp.float32),
            pltpu.VMEM((COLS, EMBED_DIM + 1), jnp.float32),
            pltpu.SemaphoreType.DMA,
            pltpu.SemaphoreType.DMA,
            pltpu.SemaphoreType.DMA,
            pltpu.SemaphoreType.DMA,
        ],
    )(_tr_kernel)(embt)

    f = functools.partial(
        pl.kernel,
        mesh=mesh,
        compiler_params=params,
        out_type=jax.ShapeDtypeStruct((BATCH,), jnp.float32),
        scratch_types=[
            pltpu.VMEM((BPW,), jnp.int32),
            pltpu.VMEM((BPW,), jnp.int32),
            pltpu.VMEM((BPW, EMBED_DIM), jnp.float32),
            pltpu.VMEM((BPW, EMBED_DIM), jnp.float32),
            pltpu.VMEM((BPW,), jnp.float32),
            pltpu.SemaphoreType.DMA,
        ],
    )(_dot_kernel)
    return f(aid1, aid2, emb_rm)


def kernel(aid1, aid2, embeddings):
    return _run(aid1.astype(jnp.int32), aid2.astype(jnp.int32), embeddings)


# speedup vs baseline: 1.1618x; 1.0541x over previous
import functools

import jax
import jax.numpy as jnp
from jax import lax
from jax.experimental import pallas as pl
from jax.experimental.pallas import tpu as pltpu
from jax.experimental.pallas import tpu_sc as plsc

N_AIDS = 1000000
EMBED_DIM = 64
BATCH = 16384

NC = 2
NS = 16
L = 16
NW = NC * NS
BPW = BATCH // NW
CH = 128
NCHUNK = BPW // CH

CB = 28
COLS = CB * L
BP = EMBED_DIM + 1
NCH_MAIN = N_AIDS // COLS
TAIL_COL = NCH_MAIN * COLS
TAIL_N = N_AIDS - TAIL_COL
NG = 35


def _transpose_blocks(a_v, b_v, lanes, nblk):
    @plsc.parallel_loop(0, nblk)
    def _(j):
        rows = j * L + lanes
        for d in range(EMBED_DIM):
            v = a_v[d, pl.ds(j * L, L)]
            plsc.store_scatter(b_v, [rows, jnp.full((L,), d, jnp.int32)], v)


def _tr_kernel(embt_hbm, out_hbm, a0, a1, b0, b1, rs0, rs1, ws0, ws1):
    wid = lax.axis_index("s") * NC + lax.axis_index("c")
    lanes = lax.iota(jnp.int32, L)
    A, B, RS, WS = (a0, a1), (b0, b1), (rs0, rs1), (ws0, ws1)

    def rd(c, u):
        return pltpu.make_async_copy(
            embt_hbm.at[:, pl.ds(c * COLS, COLS)], A[u], RS[u])

    def wr(c, u):
        return pltpu.make_async_copy(
            B[u].at[:, pl.ds(0, EMBED_DIM)],
            out_hbm.at[pl.ds(c * COLS, COLS), :], WS[u])

    for u in range(2):
        c = wid + NW * u

        @pl.when(c < NCH_MAIN)
        def _(c=c, u=u):
            rd(c, u).start()

    def gbody(g, carry):
        for u in range(2):
            i = 2 * g + u
            c = wid + NW * i

            @pl.when(c < NCH_MAIN)
            def _(c=c, u=u, g=g):
                rd(c, u).wait()

                @pl.when(g >= 1)
                def _():
                    wr(c - 2 * NW, u).wait()

                _transpose_blocks(A[u], B[u], lanes, CB)
                wr(c, u).start()

                @pl.when(c + 2 * NW < NCH_MAIN)
                def _():
                    rd(c + 2 * NW, u).start()
        return carry

    lax.fori_loop(0, NG, gbody, 0)

    k_last = (NCH_MAIN - wid + NW - 1) // NW - 1
    for u in range(2):
        li = k_last - ((k_last - u) % 2)
        wr(wid + NW * li, u).wait()

    @pl.when(wid == NW - 1)
    def _():
        pltpu.sync_copy(embt_hbm.at[:, pl.ds(TAIL_COL, TAIL_N)],
                        a0.at[:, pl.ds(0, TAIL_N)])
        _transpose_blocks(a0, b0, lanes, TAIL_N // L)
        pltpu.sync_copy(b0.at[pl.ds(0, TAIL_N), pl.ds(0, EMBED_DIM)],
                        out_hbm.at[pl.ds(TAIL_COL, TAIL_N), :])


def _dot_kernel(aid1_hbm, aid2_hbm, emb_hbm, out_hbm,
                idx1_v, idx2_v, rows1_v, rows2_v, out_v, sem):
    wid = lax.axis_index("s") * NC + lax.axis_index("c")
    base = wid * BPW

    pltpu.sync_copy(aid1_hbm.at[pl.ds(base, BPW)], idx1_v)
    pltpu.sync_copy(aid2_hbm.at[pl.ds(base, BPW)], idx2_v)

    copies = []
    for c in range(NCHUNK):
        copies.append(pltpu.async_copy(
            emb_hbm.at[idx1_v.at[pl.ds(c * CH, CH)]],
            rows1_v.at[pl.ds(c * CH, CH)], sem))
        copies.append(pltpu.async_copy(
            emb_hbm.at[idx2_v.at[pl.ds(c * CH, CH)]],
            rows2_v.at[pl.ds(c * CH, CH)], sem))
    for cp in copies:
        cp.wait()

    lanes = lax.iota(jnp.int32, L)

    def block_body(c, carry):
        rb = c * L
        row_ids = rb + lanes
        acc = jnp.zeros((L,), jnp.float32)
        for d in range(EMBED_DIM):
            col = jnp.full((L,), d, jnp.int32)
            a = plsc.load_gather(rows1_v, [row_ids, col])
            b = plsc.load_gather(rows2_v, [row_ids, col])
            acc = acc + a * b
        out_v[pl.ds(rb, L)] = acc
        return carry

    lax.fori_loop(0, BPW // L, block_body, 0)

    pltpu.sync_copy(out_v, out_hbm.at[pl.ds(base, BPW)])


@jax.jit
def _run(aid1, aid2, embeddings):
    mesh = plsc.VectorSubcoreMesh(core_axis_name="c", subcore_axis_name="s")
    params = pltpu.CompilerParams(
        needs_layout_passes=False, use_tc_tiling_on_sc=False)

    embt = embeddings.T
    emb_rm = functools.partial(
        pl.kernel,
        mesh=mesh,
        compiler_params=params,
        out_type=jax.ShapeDtypeStruct((N_AIDS, EMBED_DIM), jnp.float32),
        scratch_types=[
            pltpu.VMEM((EMBED_DIM, COLS), jnp.float32),
            pltpu.VMEM((EMBED_DIM, COLS), jnp.float32),
            pltpu.VMEM((COLS, BP), jnp.float32),
            pltpu.VMEM((COLS, BP), jnp.float32),
            pltpu.SemaphoreType.DMA,
            pltpu.SemaphoreType.DMA,
            pltpu.SemaphoreType.DMA,
            pltpu.SemaphoreType.DMA,
        ],
    )(_tr_kernel)(embt)

    f = functools.partial(
        pl.kernel,
        mesh=mesh,
        compiler_params=params,
        out_type=jax.ShapeDtypeStruct((BATCH,), jnp.float32),
        scratch_types=[
            pltpu.VMEM((BPW,), jnp.int32),
            pltpu.VMEM((BPW,), jnp.int32),
            pltpu.VMEM((BPW, EMBED_DIM), jnp.float32),
            pltpu.VMEM((BPW, EMBED_DIM), jnp.float32),
            pltpu.VMEM((BPW,), jnp.float32),
            pltpu.SemaphoreType.DMA,
        ],
    )(_dot_kernel)
    return f(aid1, aid2, emb_rm)


def kernel(aid1, aid2, embeddings):
    return _run(aid1.astype(jnp.int32), aid2.astype(jnp.int32), embeddings)
